# baseline (speedup 1.0000x reference)
import numpy as np

import jax
import jax.numpy as jnp
from jax.experimental import pallas as pl
from jax.experimental.pallas import tpu as pltpu
from jax.sharding import Mesh, PartitionSpec as P

_BLK_ROWS = 256


def _softmax_block(x_ref, o_ref):
    m = jnp.max(x_ref[...], axis=-1, keepdims=True)
    o_ref[...] = jnp.exp(x_ref[...] - m)
    e = o_ref[...]
    s = jnp.sum(e, axis=-1, keepdims=True)
    o_ref[...] = e * (1.0 / s)


def _tc_softmax(x):
    rows, k = x.shape
    return pl.pallas_call(
        _softmax_block,
        grid=(rows // _BLK_ROWS,),
        in_specs=[pl.BlockSpec((_BLK_ROWS, k), lambda i: (i, 0))],
        out_specs=pl.BlockSpec((_BLK_ROWS, k), lambda i: (i, 0)),
        out_shape=jax.ShapeDtypeStruct((rows, k), x.dtype),
        compiler_params=pltpu.CompilerParams(
            dimension_semantics=("parallel",),
        ),
    )(x)


def kernel(scores):
    b, h, q, k = scores.shape
    rows = b * h * q
    x = scores.reshape(rows, k)
    devs = jax.devices()
    n = len(devs)
    if n > 1 and rows % (n * _BLK_ROWS) == 0:
        mesh = Mesh(np.array(devs), ("d",))
        f = jax.shard_map(
            _tc_softmax,
            mesh=mesh,
            in_specs=P("d", None),
            out_specs=P("d", None),
            check_vma=False,
        )
        out = f(x)
    else:
        out = _tc_softmax(x)
    return out.reshape(b, h, q, k)


# speedup vs baseline: 6.5510x; 6.5510x over previous
import jax
import jax.numpy as jnp
from jax.experimental import pallas as pl
from jax.experimental.pallas import tpu as pltpu

_BLK_ROWS = 256


def _softmax_block(x_ref, o_ref):
    m = jnp.max(x_ref[...], axis=-1, keepdims=True)
    o_ref[...] = jnp.exp(x_ref[...] - m)
    e = o_ref[...]
    s = jnp.sum(e, axis=-1, keepdims=True)
    o_ref[...] = e * (1.0 / s)


def kernel(scores):
    b, h, q, k = scores.shape
    rows = b * h * q
    x = scores.reshape(rows, k)
    out = pl.pallas_call(
        _softmax_block,
        grid=(rows // _BLK_ROWS,),
        in_specs=[pl.BlockSpec((_BLK_ROWS, k), lambda i: (i, 0))],
        out_specs=pl.BlockSpec((_BLK_ROWS, k), lambda i: (i, 0)),
        out_shape=jax.ShapeDtypeStruct((rows, k), scores.dtype),
        compiler_params=pltpu.CompilerParams(
            dimension_semantics=("parallel",),
        ),
    )(x)
    return out.reshape(b, h, q, k)
